# baseline (speedup 1.0000x reference)
import functools

import jax
import jax.numpy as jnp
from jax import lax
from jax.experimental import pallas as pl
from jax.experimental.pallas import tpu as pltpu
from jax.experimental.pallas import tpu_sc as plsc

N, P2, W2, TOPK, CKV = 8, 49, 64, 4, 192
R = N * P2 * W2
B = R * TOPK
NWIN = N * P2
NC, NS, L = 2, 16, 16
NW = NC * NS
UWIN = NWIN // NW
MAXWIN = UWIN + 1
QB = W2 // 4
VJ = CKV // L

_mesh = plsc.VectorSubcoreMesh(core_axis_name="c", subcore_axis_name="s")


@functools.partial(
    pl.kernel,
    mesh=_mesh,
    compiler_params=pltpu.CompilerParams(use_tc_tiling_on_sc=True,
                                         needs_layout_passes=False,
                                         disable_bounds_checks=True,
                                         disable_semaphore_checks=True),
    out_type=jax.ShapeDtypeStruct((R, TOPK, CKV), jnp.float32),
    scratch_types=[
        pltpu.VMEM((MAXWIN, TOPK, W2), jnp.int32),
        [pltpu.VMEM((W2, CKV), jnp.float32)] * 3,
        [pltpu.VMEM((QB, TOPK, CKV), jnp.float32)] * 4,
        pltpu.SemaphoreType.DMA,
        [pltpu.SemaphoreType.DMA] * 3,
        [pltpu.SemaphoreType.DMA] * 4,
    ],
)
def _sc_gather(idx_hbm, kv_hbm, out_hbm, idx_v, win, wbuf, isem, ssems, wsems):
    wid = lax.axis_index("s") * NC + lax.axis_index("c")
    tail_w = NWIN - NW // 4 + (wid >> 2)
    tail_q = wid & 3

    ih = []
    for k in range(UWIN):
        ih.append(pltpu.async_copy(idx_hbm.at[wid + k * NW], idx_v.at[k], isem))
    ih.append(pltpu.async_copy(idx_hbm.at[tail_w], idx_v.at[UWIN], isem))

    def write_descr(k, q, wrow=None):
        if wrow is None:
            wrow = (wid + k * NW) * W2 + q * QB
        return pltpu.make_async_copy(wbuf[q], out_hbm.at[pl.ds(wrow, QB)],
                                     wsems[q])

    pltpu.make_async_copy(kv_hbm.at[wid], win[0], ssems[0]).start()
    pltpu.make_async_copy(kv_hbm.at[tail_w], win[2], ssems[2]).start()
    for h in ih:
        h.wait()

    lane = lax.iota(jnp.int32, L)
    tvec = lane & (TOPK - 1)

    def compact(kvec, q, qvec, wslot, wref):
        @plsc.parallel_loop(0, QB, unroll=2)
        def block(b):
            bvec = qvec * QB + b
            iv = plsc.load_gather(idx_v, [kvec, tvec, bvec])
            for t in range(TOPK):
                r = iv[t]
                for j in range(VJ):
                    wbuf[wslot][b, t, pl.ds(j * L, L)] = wref[r, pl.ds(j * L, L)]

    def do_window(k, par):
        @pl.when(k + 1 < UWIN)
        def _():
            pltpu.make_async_copy(kv_hbm.at[wid + (k + 1) * NW], win[1 - par],
                                  ssems[1 - par]).start()

        pltpu.make_async_copy(kv_hbm.at[wid], win[par], ssems[par]).wait()
        kvec = jnp.full((L,), k, jnp.int32)

        for q in range(4):
            @pl.when(k >= 1)
            def _():
                write_descr(k - 1, q).wait()

            compact(kvec, q, jnp.full((L,), q, jnp.int32), q, win[par])
            write_descr(k, q).start()

    def pair(kk, _):
        for par in range(2):
            do_window(kk * 2 + par, par)
        return 0

    lax.fori_loop(0, UWIN // 2, pair, 0)

    write_descr(UWIN - 1, 0).wait()
    pltpu.make_async_copy(kv_hbm.at[tail_w], win[2], ssems[2]).wait()
    compact(jnp.full((L,), UWIN, jnp.int32), 0,
            jnp.full((L,), tail_q, jnp.int32), 0, win[2])
    write_descr(0, 0, wrow=tail_w * W2 + tail_q * QB).start()

    for q in range(4):
        write_descr(0, q).wait()


def kernel(r_idx, r_weight, kv):
    del r_weight
    idx3 = jnp.transpose(r_idx, (0, 1, 3, 2)).reshape(NWIN, TOPK, W2)
    kv3 = kv.reshape(NWIN, W2, CKV)
    out3 = _sc_gather(idx3, kv3)
    return out3.reshape(N, P2, W2, TOPK, CKV)


# speedup vs baseline: 1.0057x; 1.0057x over previous
import functools

import jax
import jax.numpy as jnp
from jax import lax
from jax.experimental import pallas as pl
from jax.experimental.pallas import tpu as pltpu
from jax.experimental.pallas import tpu_sc as plsc

N, P2, W2, TOPK, CKV = 8, 49, 64, 4, 192
R = N * P2 * W2
B = R * TOPK
NWIN = N * P2
NC, NS, L = 2, 16, 16
NW = NC * NS
UWIN = NWIN // NW
MAXWIN = UWIN + 1
QB = W2 // 4
VJ = CKV // L

_mesh = plsc.VectorSubcoreMesh(core_axis_name="c", subcore_axis_name="s")


@functools.partial(
    pl.kernel,
    mesh=_mesh,
    compiler_params=pltpu.CompilerParams(use_tc_tiling_on_sc=True,
                                         needs_layout_passes=False),
    out_type=jax.ShapeDtypeStruct((R, TOPK, CKV), jnp.float32),
    scratch_types=[
        pltpu.VMEM((MAXWIN, TOPK, W2), jnp.int32),
        [pltpu.VMEM((W2, CKV), jnp.float32)] * 3,
        [pltpu.VMEM((QB, TOPK, CKV), jnp.float32)] * 4,
        pltpu.SemaphoreType.DMA,
        [pltpu.SemaphoreType.DMA] * 3,
        [pltpu.SemaphoreType.DMA] * 4,
    ],
)
def _sc_gather(idx_hbm, kv_hbm, out_hbm, idx_v, win, wbuf, isem, ssems, wsems):
    wid = lax.axis_index("s") * NC + lax.axis_index("c")
    tail_w = NWIN - NW // 4 + (wid >> 2)
    tail_q = wid & 3

    ih = []
    for k in range(UWIN):
        ih.append(pltpu.async_copy(idx_hbm.at[wid + k * NW], idx_v.at[k], isem))
    ih.append(pltpu.async_copy(idx_hbm.at[tail_w], idx_v.at[UWIN], isem))

    def write_descr(k, q, wrow=None):
        if wrow is None:
            wrow = (wid + k * NW) * W2 + q * QB
        return pltpu.make_async_copy(wbuf[q], out_hbm.at[pl.ds(wrow, QB)],
                                     wsems[q])

    pltpu.make_async_copy(kv_hbm.at[wid], win[0], ssems[0]).start()
    pltpu.make_async_copy(kv_hbm.at[tail_w], win[2], ssems[2]).start()
    for h in ih:
        h.wait()

    lane = lax.iota(jnp.int32, L)
    tvec = lane & (TOPK - 1)

    def compact(kvec, q, qvec, wslot, wref):
        @plsc.parallel_loop(0, QB, unroll=2)
        def block(b):
            bvec = qvec * QB + b
            iv = plsc.load_gather(idx_v, [kvec, tvec, bvec])
            for t in range(TOPK):
                r = iv[t]
                for j in range(VJ):
                    wbuf[wslot][b, t, pl.ds(j * L, L)] = wref[r, pl.ds(j * L, L)]

    def do_window(k, par):
        @pl.when(k + 1 < UWIN)
        def _():
            pltpu.make_async_copy(kv_hbm.at[wid + (k + 1) * NW], win[1 - par],
                                  ssems[1 - par]).start()

        pltpu.make_async_copy(kv_hbm.at[wid], win[par], ssems[par]).wait()
        kvec = jnp.full((L,), k, jnp.int32)

        for q in range(4):
            @pl.when(k >= 1)
            def _():
                write_descr(k - 1, q).wait()

            compact(kvec, q, jnp.full((L,), q, jnp.int32), q, win[par])
            write_descr(k, q).start()

    def pair(kk, _):
        for par in range(2):
            do_window(kk * 2 + par, par)
        return 0

    lax.fori_loop(0, UWIN // 2, pair, 0)

    write_descr(UWIN - 1, 0).wait()
    pltpu.make_async_copy(kv_hbm.at[tail_w], win[2], ssems[2]).wait()
    compact(jnp.full((L,), UWIN, jnp.int32), 0,
            jnp.full((L,), tail_q, jnp.int32), 0, win[2])
    write_descr(0, 0, wrow=tail_w * W2 + tail_q * QB).start()

    for q in range(4):
        write_descr(0, q).wait()


def kernel(r_idx, r_weight, kv):
    del r_weight
    idx3 = jnp.transpose(r_idx, (0, 1, 3, 2)).reshape(NWIN, TOPK, W2)
    kv3 = kv.reshape(NWIN, W2, CKV)
    out3 = _sc_gather(idx3, kv3)
    return out3.reshape(N, P2, W2, TOPK, CKV)
